# Initial kernel scaffold; baseline (speedup 1.0000x reference)
import functools

import jax
import jax.numpy as jnp
from jax import lax
from jax.experimental import pallas as pl
from jax.experimental.pallas import tpu as pltpu
from jax.experimental.pallas import tpu_sc as plsc

NUM_EMB_ROWS = 1000001
EMB_D = 64
BATCH = 16384
SEQ = 50

NC = 2
NS = 16
NW = NC * NS

CHUNK = 128
TOTAL = BATCH * SEQ
NCHUNKS = TOTAL // CHUNK
CH_PER_W = NCHUNKS // NW
K = 8
NGROUPS = CH_PER_W // K


@functools.partial(
    pl.kernel,
    out_type=jax.ShapeDtypeStruct((NCHUNKS, CHUNK, EMB_D), jnp.float32),
    mesh=plsc.VectorSubcoreMesh(core_axis_name="c", subcore_axis_name="s"),
    scratch_types=[
        pltpu.VMEM((K, CHUNK), jnp.int32),
        pltpu.VMEM((K, CHUNK, EMB_D), jnp.float32),
        pltpu.SemaphoreType.DMA,
    ],
)
def _emb_lookup(x_hbm, table_hbm, out_hbm, idx_v, rows_v, sem):
    wid = lax.axis_index("s") * NC + lax.axis_index("c")
    chunk0 = wid * CH_PER_W

    def group(g, carry):
        row0 = chunk0 + g * K
        pltpu.sync_copy(x_hbm.at[pl.ds(row0, K)], idx_v)
        copies = [
            pltpu.async_copy(table_hbm.at[idx_v.at[j]], rows_v.at[j], sem)
            for j in range(K)
        ]
        for c in copies:
            c.wait()
        pltpu.sync_copy(rows_v, out_hbm.at[pl.ds(row0, K)])
        return carry

    lax.fori_loop(0, NGROUPS, group, 0)


def kernel(x, table):
    x_flat = x.astype(jnp.int32).reshape(NCHUNKS, CHUNK)
    out = _emb_lookup(x_flat, table)
    return out.reshape(BATCH, SEQ, EMB_D)


# speedup vs baseline: 1.8425x; 1.8425x over previous
import functools

import jax
import jax.numpy as jnp
from jax import lax
from jax.experimental import pallas as pl
from jax.experimental.pallas import tpu as pltpu
from jax.experimental.pallas import tpu_sc as plsc

NUM_EMB_ROWS = 1000001
EMB_D = 64
BATCH = 16384
SEQ = 50

NC = 2
NS = 16
NW = NC * NS

CHUNK = 128
TOTAL = BATCH * SEQ
NCHUNKS = TOTAL // CHUNK
CH_PER_W = NCHUNKS // NW
K = 8
NGROUPS = CH_PER_W // K


@functools.partial(
    pl.kernel,
    out_type=jax.ShapeDtypeStruct((NCHUNKS, CHUNK, EMB_D), jnp.float32),
    mesh=plsc.VectorSubcoreMesh(core_axis_name="c", subcore_axis_name="s"),
    scratch_types=[
        pltpu.VMEM((K, CHUNK), jnp.int32),
        pltpu.VMEM((K, CHUNK, EMB_D), jnp.float32),
        pltpu.SemaphoreType.DMA,
    ],
    compiler_params=pltpu.CompilerParams(use_tc_tiling_on_sc=False),
)
def _emb_lookup(x_hbm, table_hbm, out_hbm, idx_v, rows_v, sem):
    wid = lax.axis_index("s") * NC + lax.axis_index("c")
    chunk0 = wid * CH_PER_W

    def group(g, carry):
        row0 = chunk0 + g * K
        pltpu.sync_copy(x_hbm.at[pl.ds(row0, K)], idx_v)
        copies = [
            pltpu.async_copy(table_hbm.at[idx_v.at[j]], rows_v.at[j], sem)
            for j in range(K)
        ]
        for c in copies:
            c.wait()
        pltpu.sync_copy(rows_v, out_hbm.at[pl.ds(row0, K)])
        return carry

    lax.fori_loop(0, NGROUPS, group, 0)


def kernel(x, table):
    x_flat = x.astype(jnp.int32).reshape(NCHUNKS, CHUNK)
    out = _emb_lookup(x_flat, table)
    return out.reshape(BATCH, SEQ, EMB_D)


# speedup vs baseline: 1.8651x; 1.0122x over previous
import functools

import jax
import jax.numpy as jnp
from jax import lax
from jax.experimental import pallas as pl
from jax.experimental.pallas import tpu as pltpu
from jax.experimental.pallas import tpu_sc as plsc

NUM_EMB_ROWS = 1000001
EMB_D = 64
BATCH = 16384
SEQ = 50

NC = 2
NS = 16
NW = NC * NS

CHUNK = 128
TOTAL = BATCH * SEQ
NCHUNKS = TOTAL // CHUNK
CH_PER_W = NCHUNKS // NW
NBUF = 4


@functools.partial(
    pl.kernel,
    out_type=jax.ShapeDtypeStruct((NCHUNKS, CHUNK, EMB_D), jnp.float32),
    mesh=plsc.VectorSubcoreMesh(core_axis_name="c", subcore_axis_name="s"),
    scratch_types=[
        pltpu.VMEM((CH_PER_W, CHUNK), jnp.int32),
        pltpu.VMEM((NBUF, CHUNK, EMB_D), jnp.float32),
        [pltpu.SemaphoreType.DMA] * NBUF,
        [pltpu.SemaphoreType.DMA] * NBUF,
    ],
    compiler_params=pltpu.CompilerParams(use_tc_tiling_on_sc=False),
)
def _emb_lookup(x_hbm, table_hbm, out_hbm, idx_all, rows, sg, so):
    wid = lax.axis_index("s") * NC + lax.axis_index("c")
    chunk0 = wid * CH_PER_W
    pltpu.sync_copy(x_hbm.at[pl.ds(chunk0, CH_PER_W)], idx_all)

    def fire_g(g, b):
        pltpu.async_copy(table_hbm.at[idx_all.at[g]], rows.at[b], sg[b])

    def drain_g(b):
        pltpu.make_async_copy(table_hbm.at[idx_all.at[0]], rows.at[b], sg[b]).wait()

    def fire_w(g, b):
        pltpu.async_copy(rows.at[b], out_hbm.at[chunk0 + g], so[b])

    def drain_w(b):
        pltpu.make_async_copy(rows.at[b], out_hbm.at[chunk0], so[b]).wait()

    fire_g(0, 0)
    fire_g(1, 1)
    drain_g(0); fire_w(0, 0); fire_g(2, 2)
    drain_g(1); fire_w(1, 1); fire_g(3, 3)

    def loop_body(t, carry):
        g0 = 2 + t * NBUF
        for i in range(NBUF):
            g = g0 + i
            b = (2 + i) % NBUF
            bn = (b + 2) % NBUF
            drain_g(b)
            fire_w(g, b)
            drain_w(bn)
            fire_g(g + 2, bn)
        return carry

    lax.fori_loop(0, (CH_PER_W - NBUF) // NBUF, loop_body, 0)

    drain_g(2); fire_w(CH_PER_W - 2, 2); drain_w(0)
    drain_g(3); fire_w(CH_PER_W - 1, 3); drain_w(1)
    drain_w(2)
    drain_w(3)


def kernel(x, table):
    x_flat = x.astype(jnp.int32).reshape(NCHUNKS, CHUNK)
    out = _emb_lookup(x_flat, table)
    return out.reshape(BATCH, SEQ, EMB_D)


# speedup vs baseline: 1.8752x; 1.0054x over previous
import functools

import jax
import jax.numpy as jnp
from jax import lax
from jax.experimental import pallas as pl
from jax.experimental.pallas import tpu as pltpu
from jax.experimental.pallas import tpu_sc as plsc

NUM_EMB_ROWS = 1000001
EMB_D = 64
BATCH = 16384
SEQ = 50

NC = 2
NS = 16
NW = NC * NS

CHUNK = 128
TOTAL = BATCH * SEQ
NCHUNKS = TOTAL // CHUNK
CH_PER_W = NCHUNKS // NW
NBUF = 8
AHEAD = 4


@functools.partial(
    pl.kernel,
    out_type=jax.ShapeDtypeStruct((NCHUNKS, CHUNK, EMB_D), jnp.float32),
    mesh=plsc.VectorSubcoreMesh(core_axis_name="c", subcore_axis_name="s"),
    scratch_types=[
        pltpu.VMEM((CH_PER_W, CHUNK), jnp.int32),
        pltpu.VMEM((NBUF, CHUNK, EMB_D), jnp.float32),
        [pltpu.SemaphoreType.DMA] * NBUF,
        [pltpu.SemaphoreType.DMA] * NBUF,
    ],
    compiler_params=pltpu.CompilerParams(use_tc_tiling_on_sc=False),
)
def _emb_lookup(x_hbm, table_hbm, out_hbm, idx_all, rows, sg, so):
    wid = lax.axis_index("s") * NC + lax.axis_index("c")
    chunk0 = wid * CH_PER_W
    pltpu.sync_copy(x_hbm.at[pl.ds(chunk0, CH_PER_W)], idx_all)

    def fire_g(g, b):
        pltpu.async_copy(table_hbm.at[idx_all.at[g]], rows.at[b], sg[b])

    def drain_g(b):
        pltpu.make_async_copy(table_hbm.at[idx_all.at[0]], rows.at[b], sg[b]).wait()

    def fire_w(g, b):
        pltpu.async_copy(rows.at[b], out_hbm.at[chunk0 + g], so[b])

    def drain_w(b):
        pltpu.make_async_copy(rows.at[b], out_hbm.at[chunk0], so[b]).wait()

    for g in range(AHEAD):
        fire_g(g, g % NBUF)
    for g in range(NBUF - AHEAD):
        drain_g(g % NBUF)
        fire_w(g, g % NBUF)
        fire_g(g + AHEAD, (g + AHEAD) % NBUF)

    start = NBUF - AHEAD

    def loop_body(t, carry):
        g0 = start + t * NBUF
        for i in range(NBUF):
            g = g0 + i
            b = (start + i) % NBUF
            bn = (b + AHEAD) % NBUF
            drain_g(b)
            fire_w(g, b)
            drain_w(bn)
            fire_g(g + AHEAD, bn)
        return carry

    lax.fori_loop(0, (CH_PER_W - NBUF) // NBUF, loop_body, 0)

    for g in range(CH_PER_W - AHEAD, CH_PER_W):
        b = g % NBUF
        drain_g(b)
        fire_w(g, b)
        drain_w((b + AHEAD) % NBUF)
    for g in range(CH_PER_W - (NBUF - AHEAD), CH_PER_W):
        drain_w(g % NBUF)


def kernel(x, table):
    x_flat = x.astype(jnp.int32).reshape(NCHUNKS, CHUNK)
    out = _emb_lookup(x_flat, table)
    return out.reshape(BATCH, SEQ, EMB_D)


# speedup vs baseline: 2.4844x; 1.3249x over previous
import functools

import jax
import jax.numpy as jnp
from jax import lax
from jax.experimental import pallas as pl
from jax.experimental.pallas import tpu as pltpu
from jax.experimental.pallas import tpu_sc as plsc

NUM_EMB_ROWS = 1000001
EMB_D = 64
BATCH = 16384
SEQ = 50
SEQP = 56

NC = 2
NS = 16
NW = NC * NS

B_PER_W = BATCH // NW
BSTEP = 2
STEPS = B_PER_W // BSTEP
ROWS_STEP = BSTEP * SEQP
NBUF = 8
AHEAD = 4

OUT_ROWS = BATCH * SEQP


@functools.partial(
    pl.kernel,
    out_type=jax.ShapeDtypeStruct((OUT_ROWS, 128), jnp.float32),
    mesh=plsc.VectorSubcoreMesh(core_axis_name="c", subcore_axis_name="s"),
    scratch_types=[
        pltpu.VMEM((B_PER_W * SEQP,), jnp.int32),
        pltpu.VMEM((NBUF, ROWS_STEP, EMB_D), jnp.float32),
        [pltpu.SemaphoreType.DMA] * NBUF,
        [pltpu.SemaphoreType.DMA] * NBUF,
    ],
    compiler_params=pltpu.CompilerParams(use_tc_tiling_on_sc=False),
)
def _emb_lookup(x_hbm, table_hbm, out_hbm, idx_all, rows, sg, so):
    wid = lax.axis_index("s") * NC + lax.axis_index("c")
    idx0 = wid * (B_PER_W * SEQP)
    row0 = wid * (B_PER_W * SEQP)
    pltpu.sync_copy(x_hbm.at[pl.ds(idx0, B_PER_W * SEQP)], idx_all)

    def fire_g(g, b):
        for u in range(BSTEP):
            pltpu.async_copy(
                table_hbm.at[idx_all.at[pl.ds((g * BSTEP + u) * SEQP, SEQ)]],
                rows.at[b, pl.ds(u * SEQP, SEQ)],
                sg[b],
            )

    def drain_g(b):
        for u in range(BSTEP):
            pltpu.make_async_copy(
                table_hbm.at[idx_all.at[pl.ds(u * SEQP, SEQ)]],
                rows.at[b, pl.ds(u * SEQP, SEQ)],
                sg[b],
            ).wait()

    def fire_w(g, b):
        pltpu.async_copy(
            rows.at[b], out_hbm.at[pl.ds(row0 + g * ROWS_STEP, ROWS_STEP), 0:EMB_D],
            so[b],
        )

    def drain_w(b):
        pltpu.make_async_copy(
            rows.at[b], out_hbm.at[pl.ds(row0, ROWS_STEP), 0:EMB_D], so[b]
        ).wait()

    for g in range(AHEAD):
        fire_g(g, g % NBUF)
    for g in range(NBUF - AHEAD):
        drain_g(g % NBUF)
        fire_w(g, g % NBUF)
        fire_g(g + AHEAD, (g + AHEAD) % NBUF)

    start = NBUF - AHEAD

    def loop_body(t, carry):
        g0 = start + t * NBUF
        for i in range(NBUF):
            g = g0 + i
            b = (start + i) % NBUF
            bn = (b + AHEAD) % NBUF
            drain_g(b)
            fire_w(g, b)
            drain_w(bn)
            fire_g(g + AHEAD, bn)
        return carry

    lax.fori_loop(0, (STEPS - NBUF) // NBUF, loop_body, 0)

    for g in range(STEPS - AHEAD, STEPS):
        b = g % NBUF
        drain_g(b)
        fire_w(g, b)
        drain_w((b + AHEAD) % NBUF)
    for g in range(STEPS - (NBUF - AHEAD), STEPS):
        drain_w(g % NBUF)


def kernel(x, table):
    x56 = jnp.pad(x.astype(jnp.int32), ((0, 0), (0, SEQP - SEQ))).reshape(-1)
    out = _emb_lookup(x56, table)
    return out.reshape(BATCH, SEQP, 128)[:, :SEQ, :EMB_D]
